# baseline (device time: 13307 ns/iter reference)
import jax
import jax.numpy as jnp
from jax import lax
from jax.experimental import pallas as pl
from jax.experimental.pallas import tpu as pltpu

N_DEV = 4
ROW_CHUNKS = 8


def kernel(x):
    m_per, n = x.shape
    m_global = m_per * N_DEV
    m_blk = m_per // ROW_CHUNKS

    def body(x_ref, out_ref, bufs, own_ref, comm_ref,
             copy_sems, send_sems, recv_sems):
        my_pos = lax.axis_index("i")

        def chunk_copy(i):
            return pltpu.make_async_copy(
                x_ref.at[pl.ds(i * m_blk, m_blk), :],
                bufs.at[i % 2],
                copy_sems.at[i % 2],
            )

        chunk_copy(0).start()
        barrier_sem = pltpu.get_barrier_semaphore()
        for off in range(1, N_DEV):
            pl.semaphore_signal(
                barrier_sem, inc=1,
                device_id=((my_pos + off) % N_DEV,),
                device_id_type=pl.DeviceIdType.MESH,
            )
        pl.semaphore_wait(barrier_sem, N_DEV - 1)

        acc = jnp.zeros((1, n), jnp.float32)
        for i in range(ROW_CHUNKS):
            if i + 1 < ROW_CHUNKS:
                chunk_copy(i + 1).start()
            chunk_copy(i).wait()
            acc = acc + jnp.sum(bufs[i % 2], axis=0, keepdims=True)
        own_ref[:, :] = acc

        sends = []
        for off in range(1, N_DEV):
            rdma = pltpu.make_async_remote_copy(
                src_ref=own_ref,
                dst_ref=comm_ref.at[3 - off],
                send_sem=send_sems.at[off - 1],
                recv_sem=recv_sems.at[3 - off],
                device_id=((my_pos + off) % N_DEV,),
                device_id_type=pl.DeviceIdType.MESH,
            )
            rdma.start()
            sends.append(rdma)

        for off in range(1, N_DEV):
            slot = off - 1
            recv = pltpu.make_async_remote_copy(
                src_ref=own_ref,
                dst_ref=comm_ref.at[slot],
                send_sem=send_sems.at[off - 1],
                recv_sem=recv_sems.at[slot],
                device_id=((my_pos + off) % N_DEV,),
                device_id_type=pl.DeviceIdType.MESH,
            )
            recv.wait_recv()
            acc = acc + comm_ref[slot, :, :]

        out_ref[:, :] = acc * (1.0 / m_global)

        for rdma in sends:
            rdma.wait_send()

    return pl.pallas_call(
        body,
        out_shape=jax.ShapeDtypeStruct((1, n), jnp.float32),
        in_specs=[pl.BlockSpec(memory_space=pl.ANY)],
        out_specs=pl.BlockSpec(memory_space=pltpu.VMEM),
        scratch_shapes=[
            pltpu.VMEM((2, m_blk, n), jnp.float32),
            pltpu.VMEM((1, n), jnp.float32),
            pltpu.VMEM((N_DEV - 1, 1, n), jnp.float32),
            pltpu.SemaphoreType.DMA((2,)),
            pltpu.SemaphoreType.DMA((N_DEV - 1,)),
            pltpu.SemaphoreType.DMA((N_DEV - 1,)),
        ],
        compiler_params=pltpu.CompilerParams(collective_id=0),
    )(x)


# device time: 12818 ns/iter; 1.0381x vs baseline; 1.0381x over previous
import jax
import jax.numpy as jnp
from jax import lax
from jax.experimental import pallas as pl
from jax.experimental.pallas import tpu as pltpu

N_DEV = 4
ROW_CHUNKS = 8


def kernel(x):
    m_per, n = x.shape
    m_global = m_per * N_DEV
    m_blk = m_per // ROW_CHUNKS

    def body(x_ref, out_ref, bufs, own_ref, comm_ref,
             copy_sems, send_sems, recv_sems):
        my_pos = lax.axis_index("i")

        def chunk_copy(i):
            return pltpu.make_async_copy(
                x_ref.at[pl.ds(i * m_blk, m_blk), :],
                bufs.at[i % 2],
                copy_sems.at[i % 2],
            )

        chunk_copy(0).start()
        barrier_sem = pltpu.get_barrier_semaphore()
        for off in range(1, N_DEV):
            pl.semaphore_signal(
                barrier_sem, inc=1,
                device_id=((my_pos + off) % N_DEV,),
                device_id_type=pl.DeviceIdType.MESH,
            )
        pl.semaphore_wait(barrier_sem, N_DEV - 1)

        acc = jnp.zeros((1, n), jnp.float32)
        for i in range(ROW_CHUNKS):
            if i + 1 < ROW_CHUNKS:
                chunk_copy(i + 1).start()
            chunk_copy(i).wait()
            acc = acc + jnp.sum(bufs[i % 2], axis=0, keepdims=True)
        own_ref[:, :] = acc

        sends = []
        for off in range(1, N_DEV):
            rdma = pltpu.make_async_remote_copy(
                src_ref=own_ref,
                dst_ref=comm_ref.at[3 - off],
                send_sem=send_sems.at[off - 1],
                recv_sem=recv_sems.at[3 - off],
                device_id=((my_pos + off) % N_DEV,),
                device_id_type=pl.DeviceIdType.MESH,
            )
            rdma.start()
            sends.append(rdma)

        for off in range(1, N_DEV):
            slot = off - 1
            recv = pltpu.make_async_remote_copy(
                src_ref=own_ref,
                dst_ref=comm_ref.at[slot],
                send_sem=send_sems.at[off - 1],
                recv_sem=recv_sems.at[slot],
                device_id=((my_pos + off) % N_DEV,),
                device_id_type=pl.DeviceIdType.MESH,
            )
            recv.wait_recv()
            acc = acc + comm_ref[slot, :, :]

        out_ref[:, :] = acc * (1.0 / m_global)

        for rdma in sends:
            rdma.wait_send()

    return pl.pallas_call(
        body,
        out_shape=jax.ShapeDtypeStruct((1, n), jnp.float32),
        in_specs=[pl.BlockSpec(memory_space=pl.ANY)],
        out_specs=pl.BlockSpec(memory_space=pltpu.VMEM),
        scratch_shapes=[
            pltpu.VMEM((2, m_blk, n), jnp.float32),
            pltpu.VMEM((1, n), jnp.float32),
            pltpu.VMEM((N_DEV - 1, 1, n), jnp.float32),
            pltpu.SemaphoreType.DMA((2,)),
            pltpu.SemaphoreType.DMA((N_DEV - 1,)),
            pltpu.SemaphoreType.DMA((N_DEV - 1,)),
        ],
        compiler_params=pltpu.CompilerParams(collective_id=0),
    )(pltpu.with_memory_space_constraint(x, pltpu.MemorySpace.HBM))


# device time: 11941 ns/iter; 1.1144x vs baseline; 1.0734x over previous
import jax
import jax.numpy as jnp
from jax import lax
from jax.experimental import pallas as pl
from jax.experimental.pallas import tpu as pltpu

N_DEV = 4
ROW_CHUNKS = 16
N_BUF = 4


def kernel(x):
    m_per, n = x.shape
    m_global = m_per * N_DEV
    m_blk = m_per // ROW_CHUNKS

    def body(x_ref, out_ref, bufs, own_ref, comm_ref, res_ref,
             copy_sems, out_sem, send_sems, recv_sems):
        my_pos = lax.axis_index("i")

        def chunk_copy(i):
            return pltpu.make_async_copy(
                x_ref.at[pl.ds(i * m_blk, m_blk), :],
                bufs.at[i % N_BUF],
                copy_sems.at[i % N_BUF],
            )

        for i in range(N_BUF):
            chunk_copy(i).start()
        barrier_sem = pltpu.get_barrier_semaphore()
        for off in range(1, N_DEV):
            pl.semaphore_signal(
                barrier_sem, inc=1,
                device_id=((my_pos + off) % N_DEV,),
                device_id_type=pl.DeviceIdType.MESH,
            )
        pl.semaphore_wait(barrier_sem, N_DEV - 1)

        acc = jnp.zeros((1, n), jnp.float32)
        for i in range(ROW_CHUNKS):
            chunk_copy(i).wait()
            acc = acc + jnp.sum(bufs[i % N_BUF], axis=0, keepdims=True)
            if i + N_BUF < ROW_CHUNKS:
                chunk_copy(i + N_BUF).start()
        own_ref[:, :] = acc

        sends = []
        for off in (2, 1, 3):
            rdma = pltpu.make_async_remote_copy(
                src_ref=own_ref,
                dst_ref=comm_ref.at[3 - off],
                send_sem=send_sems.at[off - 1],
                recv_sem=recv_sems.at[3 - off],
                device_id=((my_pos + off) % N_DEV,),
                device_id_type=pl.DeviceIdType.MESH,
            )
            rdma.start()
            sends.append(rdma)

        for off in range(1, N_DEV):
            slot = off - 1
            recv = pltpu.make_async_remote_copy(
                src_ref=own_ref,
                dst_ref=comm_ref.at[slot],
                send_sem=send_sems.at[off - 1],
                recv_sem=recv_sems.at[slot],
                device_id=((my_pos + off) % N_DEV,),
                device_id_type=pl.DeviceIdType.MESH,
            )
            recv.wait_recv()
            acc = acc + comm_ref[slot, :, :]

        res_ref[:, :] = acc * (1.0 / m_global)
        out_cp = pltpu.make_async_copy(res_ref, out_ref, out_sem)
        out_cp.start()
        out_cp.wait()

        for rdma in sends:
            rdma.wait_send()

    return pl.pallas_call(
        body,
        out_shape=jax.ShapeDtypeStruct((1, n), jnp.float32),
        in_specs=[pl.BlockSpec(memory_space=pl.ANY)],
        out_specs=pl.BlockSpec(memory_space=pl.ANY),
        scratch_shapes=[
            pltpu.VMEM((N_BUF, m_blk, n), jnp.float32),
            pltpu.VMEM((1, n), jnp.float32),
            pltpu.VMEM((N_DEV - 1, 1, n), jnp.float32),
            pltpu.VMEM((1, n), jnp.float32),
            pltpu.SemaphoreType.DMA((N_BUF,)),
            pltpu.SemaphoreType.DMA,
            pltpu.SemaphoreType.DMA((N_DEV - 1,)),
            pltpu.SemaphoreType.DMA((N_DEV - 1,)),
        ],
        compiler_params=pltpu.CompilerParams(collective_id=0),
    )(pltpu.with_memory_space_constraint(x, pltpu.MemorySpace.HBM))


# device time: 11536 ns/iter; 1.1535x vs baseline; 1.0351x over previous
import jax
import jax.numpy as jnp
from jax import lax
from jax.experimental import pallas as pl
from jax.experimental.pallas import tpu as pltpu

N_DEV = 4
ROW_CHUNKS = 4
N_BUF = 2


def kernel(x):
    m_per, n = x.shape
    m_global = m_per * N_DEV
    m_blk = m_per // ROW_CHUNKS

    def body(x_ref, out_ref, bufs, own_ref, comm_ref, res_ref,
             copy_sems, out_sem, send_sems, recv_sems):
        my_pos = lax.axis_index("i")

        def chunk_copy(i):
            return pltpu.make_async_copy(
                x_ref.at[pl.ds(i * m_blk, m_blk), :],
                bufs.at[i % N_BUF],
                copy_sems.at[i % N_BUF],
            )

        for i in range(N_BUF):
            chunk_copy(i).start()
        barrier_sem = pltpu.get_barrier_semaphore()
        for off in range(1, N_DEV):
            pl.semaphore_signal(
                barrier_sem, inc=1,
                device_id=((my_pos + off) % N_DEV,),
                device_id_type=pl.DeviceIdType.MESH,
            )
        pl.semaphore_wait(barrier_sem, N_DEV - 1)

        acc = jnp.zeros((1, n), jnp.float32)
        for i in range(ROW_CHUNKS):
            chunk_copy(i).wait()
            acc = acc + jnp.sum(bufs[i % N_BUF], axis=0, keepdims=True)
            if i + N_BUF < ROW_CHUNKS:
                chunk_copy(i + N_BUF).start()
        own_ref[:, :] = acc

        sends = []
        for off in (2, 1, 3):
            rdma = pltpu.make_async_remote_copy(
                src_ref=own_ref,
                dst_ref=comm_ref.at[3 - off],
                send_sem=send_sems.at[off - 1],
                recv_sem=recv_sems.at[3 - off],
                device_id=((my_pos + off) % N_DEV,),
                device_id_type=pl.DeviceIdType.MESH,
            )
            rdma.start()
            sends.append(rdma)

        for off in range(1, N_DEV):
            slot = off - 1
            recv = pltpu.make_async_remote_copy(
                src_ref=own_ref,
                dst_ref=comm_ref.at[slot],
                send_sem=send_sems.at[off - 1],
                recv_sem=recv_sems.at[slot],
                device_id=((my_pos + off) % N_DEV,),
                device_id_type=pl.DeviceIdType.MESH,
            )
            recv.wait_recv()
            acc = acc + comm_ref[slot, :, :]

        res_ref[:, :] = acc * (1.0 / m_global)
        out_cp = pltpu.make_async_copy(res_ref, out_ref, out_sem)
        out_cp.start()
        out_cp.wait()

        for rdma in sends:
            rdma.wait_send()

    return pl.pallas_call(
        body,
        out_shape=jax.ShapeDtypeStruct((1, n), jnp.float32),
        in_specs=[pl.BlockSpec(memory_space=pl.ANY)],
        out_specs=pl.BlockSpec(memory_space=pl.ANY),
        scratch_shapes=[
            pltpu.VMEM((N_BUF, m_blk, n), jnp.float32),
            pltpu.VMEM((1, n), jnp.float32),
            pltpu.VMEM((N_DEV - 1, 1, n), jnp.float32),
            pltpu.VMEM((1, n), jnp.float32),
            pltpu.SemaphoreType.DMA((N_BUF,)),
            pltpu.SemaphoreType.DMA,
            pltpu.SemaphoreType.DMA((N_DEV - 1,)),
            pltpu.SemaphoreType.DMA((N_DEV - 1,)),
        ],
        compiler_params=pltpu.CompilerParams(collective_id=0),
    )(pltpu.with_memory_space_constraint(x, pltpu.MemorySpace.HBM))


# device time: 11157 ns/iter; 1.1927x vs baseline; 1.0340x over previous
import jax
import jax.numpy as jnp
from jax import lax
from jax.experimental import pallas as pl
from jax.experimental.pallas import tpu as pltpu

N_DEV = 4
CHUNK_ROWS = (1312, 1312, 1312, 160)
N_BUF = 2


def kernel(x):
    m_per, n = x.shape
    m_global = m_per * N_DEV
    assert sum(CHUNK_ROWS) == m_per
    starts = [sum(CHUNK_ROWS[:i]) for i in range(len(CHUNK_ROWS))]
    max_blk = max(CHUNK_ROWS)

    def body(x_ref, out_ref, bufs, own_ref, comm_ref,
             copy_sems, send_sems, recv_sems):
        my_pos = lax.axis_index("i")

        def chunk_copy(i):
            rows = CHUNK_ROWS[i]
            return pltpu.make_async_copy(
                x_ref.at[pl.ds(starts[i], rows), :],
                bufs.at[i % N_BUF, pl.ds(0, rows), :],
                copy_sems.at[i % N_BUF],
            )

        for i in range(N_BUF):
            chunk_copy(i).start()
        barrier_sem = pltpu.get_barrier_semaphore()
        for off in range(1, N_DEV):
            pl.semaphore_signal(
                barrier_sem, inc=1,
                device_id=((my_pos + off) % N_DEV,),
                device_id_type=pl.DeviceIdType.MESH,
            )
        pl.semaphore_wait(barrier_sem, N_DEV - 1)

        acc = jnp.zeros((1, n), jnp.float32)
        for i in range(len(CHUNK_ROWS)):
            chunk_copy(i).wait()
            blk = bufs[i % N_BUF, 0:CHUNK_ROWS[i], :]
            acc = acc + jnp.sum(blk, axis=0, keepdims=True)
            if i + N_BUF < len(CHUNK_ROWS):
                chunk_copy(i + N_BUF).start()
        own_ref[:, :] = acc

        sends = []
        for off in (2, 1, 3):
            rdma = pltpu.make_async_remote_copy(
                src_ref=own_ref,
                dst_ref=comm_ref.at[3 - off],
                send_sem=send_sems.at[off - 1],
                recv_sem=recv_sems.at[3 - off],
                device_id=((my_pos + off) % N_DEV,),
                device_id_type=pl.DeviceIdType.MESH,
            )
            rdma.start()
            sends.append(rdma)

        for off in range(1, N_DEV):
            slot = off - 1
            recv = pltpu.make_async_remote_copy(
                src_ref=own_ref,
                dst_ref=comm_ref.at[slot],
                send_sem=send_sems.at[off - 1],
                recv_sem=recv_sems.at[slot],
                device_id=((my_pos + off) % N_DEV,),
                device_id_type=pl.DeviceIdType.MESH,
            )
            recv.wait_recv()
            acc = acc + comm_ref[slot, :, :]

        out_ref[:, :] = acc * (1.0 / m_global)

        for rdma in sends:
            rdma.wait_send()

    return pl.pallas_call(
        body,
        out_shape=jax.ShapeDtypeStruct((1, n), jnp.float32),
        in_specs=[pl.BlockSpec(memory_space=pl.ANY)],
        out_specs=pl.BlockSpec(memory_space=pltpu.MemorySpace.VMEM),
        scratch_shapes=[
            pltpu.VMEM((N_BUF, max_blk, n), jnp.float32),
            pltpu.VMEM((1, n), jnp.float32),
            pltpu.VMEM((N_DEV - 1, 1, n), jnp.float32),
            pltpu.SemaphoreType.DMA((N_BUF,)),
            pltpu.SemaphoreType.DMA((N_DEV - 1,)),
            pltpu.SemaphoreType.DMA((N_DEV - 1,)),
        ],
        compiler_params=pltpu.CompilerParams(collective_id=0),
    )(pltpu.with_memory_space_constraint(x, pltpu.MemorySpace.HBM))


# device time: 11079 ns/iter; 1.2011x vs baseline; 1.0070x over previous
import jax
import jax.numpy as jnp
from jax import lax
from jax.experimental import pallas as pl
from jax.experimental.pallas import tpu as pltpu

N_DEV = 4
CHUNK_ROWS = (576, 576, 576, 576, 576, 576, 576, 64)
N_BUF = 4


def kernel(x):
    m_per, n = x.shape
    m_global = m_per * N_DEV
    assert sum(CHUNK_ROWS) == m_per
    starts = [sum(CHUNK_ROWS[:i]) for i in range(len(CHUNK_ROWS))]
    max_blk = max(CHUNK_ROWS)

    def body(x_ref, out_ref, bufs, own_ref, comm_ref,
             copy_sems, send_sems, recv_sems):
        my_pos = lax.axis_index("i")

        def chunk_copy(i):
            rows = CHUNK_ROWS[i]
            return pltpu.make_async_copy(
                x_ref.at[pl.ds(starts[i], rows), :],
                bufs.at[i % N_BUF, pl.ds(0, rows), :],
                copy_sems.at[i % N_BUF],
            )

        for i in range(N_BUF):
            chunk_copy(i).start()
        barrier_sem = pltpu.get_barrier_semaphore()
        for off in range(1, N_DEV):
            pl.semaphore_signal(
                barrier_sem, inc=1,
                device_id=((my_pos + off) % N_DEV,),
                device_id_type=pl.DeviceIdType.MESH,
            )
        pl.semaphore_wait(barrier_sem, N_DEV - 1)

        acc = jnp.zeros((1, n), jnp.float32)
        for i in range(len(CHUNK_ROWS)):
            chunk_copy(i).wait()
            blk = bufs[i % N_BUF, 0:CHUNK_ROWS[i], :]
            acc = acc + jnp.sum(blk, axis=0, keepdims=True)
            if i + N_BUF < len(CHUNK_ROWS):
                chunk_copy(i + N_BUF).start()
        own_ref[:, :] = acc

        sends = []
        for off in (2, 1, 3):
            rdma = pltpu.make_async_remote_copy(
                src_ref=own_ref,
                dst_ref=comm_ref.at[3 - off],
                send_sem=send_sems.at[off - 1],
                recv_sem=recv_sems.at[3 - off],
                device_id=((my_pos + off) % N_DEV,),
                device_id_type=pl.DeviceIdType.MESH,
            )
            rdma.start()
            sends.append(rdma)

        for off in range(1, N_DEV):
            slot = off - 1
            recv = pltpu.make_async_remote_copy(
                src_ref=own_ref,
                dst_ref=comm_ref.at[slot],
                send_sem=send_sems.at[off - 1],
                recv_sem=recv_sems.at[slot],
                device_id=((my_pos + off) % N_DEV,),
                device_id_type=pl.DeviceIdType.MESH,
            )
            recv.wait_recv()
            acc = acc + comm_ref[slot, :, :]

        out_ref[:, :] = acc * (1.0 / m_global)

        for rdma in sends:
            rdma.wait_send()

    return pl.pallas_call(
        body,
        out_shape=jax.ShapeDtypeStruct((1, n), jnp.float32),
        in_specs=[pl.BlockSpec(memory_space=pl.ANY)],
        out_specs=pl.BlockSpec(memory_space=pltpu.MemorySpace.VMEM),
        scratch_shapes=[
            pltpu.VMEM((N_BUF, max_blk, n), jnp.float32),
            pltpu.VMEM((1, n), jnp.float32),
            pltpu.VMEM((N_DEV - 1, 1, n), jnp.float32),
            pltpu.SemaphoreType.DMA((N_BUF,)),
            pltpu.SemaphoreType.DMA((N_DEV - 1,)),
            pltpu.SemaphoreType.DMA((N_DEV - 1,)),
        ],
        compiler_params=pltpu.CompilerParams(collective_id=0),
    )(pltpu.with_memory_space_constraint(x, pltpu.MemorySpace.HBM))
